# Initial kernel scaffold; baseline (speedup 1.0000x reference)
import jax
import jax.numpy as jnp
from jax.experimental import pallas as pl


def kernel(xyz, features, knn_indices, Wwn1, bwn1, Wlin1, blin1, Wwn2, bwn2, Wlin2, blin2, Wfc, bfc):
    raise NotImplementedError("write your pallas kernel here")


# speedup vs baseline: 659.8280x; 659.8280x over previous
import functools

import jax
import jax.numpy as jnp
from jax import lax
from jax.experimental import pallas as pl
from jax.experimental.pallas import tpu as pltpu
from jax.experimental.pallas import tpu_sc as plsc

N = 10000
K = 32
NWORK = 32
NP_PAD = 10240
PPW = NP_PAD // NWORK
PTS = 4
EPC = PTS * K
NCHUNK = PPW // PTS


def _lrelu(x):
    return jnp.maximum(x, 0.1 * x)


def _dot(a, b):
    return jnp.dot(a, b, preferred_element_type=jnp.float32,
                   precision=lax.Precision.HIGHEST)


def _prep1_body(xt_ref, xyzt_ref, wlin1t_ref, blin1_ref, wwn1t_ref, bwn1_ref,
                wwn2t_ref, bwn2_ref, t1_ref, s1_ref, a2_ref, s2_ref):
    f1 = _lrelu(_dot(xt_ref[...], wlin1t_ref[...]) + blin1_ref[...])
    a1 = _dot(xyzt_ref[...], wwn1t_ref[...])
    t1_ref[:, :128] = f1 * (1.0 / K)
    t1_ref[:, 128:] = a1
    s1_ref[...] = a1 - bwn1_ref[...]
    a2 = _dot(xyzt_ref[...], wwn2t_ref[...])
    a2_ref[...] = a2
    s2_ref[...] = a2 - bwn2_ref[...]


def _prep2_body(x_ref, wlin2t_ref, blin2_ref, a2_ref, t2_ref):
    f2 = _lrelu(_dot(x_ref[...], wlin2t_ref[...]) + blin2_ref[...])
    t2_ref[:, :64] = f2 * (1.0 / K)
    t2_ref[:, 64:] = a2_ref[...]


def _final_body(x_ref, wfct_ref, bfc_ref, r_ref):
    r_ref[...] = _dot(x_ref[...], wfct_ref[...]) + bfc_ref[...]


def _make_sc_layer(C):
    G = C // 16
    mesh = plsc.VectorSubcoreMesh(core_axis_name="c", subcore_axis_name="s")

    @functools.partial(
        pl.kernel,
        mesh=mesh,
        out_type=jax.ShapeDtypeStruct((NP_PAD, C), jnp.float32),
        scratch_types=[
            pltpu.VMEM((EPC,), jnp.int32),
            pltpu.VMEM((EPC, 2 * C), jnp.float32),
            pltpu.VMEM((PTS, C), jnp.float32),
            pltpu.VMEM((PTS, C), jnp.float32),
            pltpu.SemaphoreType.DMA,
        ],
    )
    def sc_layer(t_hbm, idx_hbm, s_hbm, out_hbm, idx_v, rows_v, s_v, o_v, sem):
        wid = lax.axis_index("s") * 2 + lax.axis_index("c")
        base_pt = wid * PPW

        @pl.loop(0, NCHUNK)
        def _chunk(ch):
            p0 = base_pt + ch * PTS
            pltpu.sync_copy(idx_hbm.at[pl.ds(p0 * K, EPC)], idx_v)
            pltpu.async_copy(t_hbm.at[idx_v], rows_v, sem).wait()
            pltpu.sync_copy(s_hbm.at[pl.ds(p0, PTS)], s_v)
            for p in range(PTS):
                svs = [s_v[p, pl.ds(g * 16, 16)] for g in range(G)]

                def body(k, accs, p=p, svs=svs):
                    e = p * K + k
                    out = []
                    for g in range(G):
                        a = rows_v[e, pl.ds(C + g * 16, 16)]
                        f = rows_v[e, pl.ds(g * 16, 16)]
                        w = a - svs[g]
                        w = jnp.maximum(w, 0.1 * w)
                        out.append(accs[g] + w * f)
                    return tuple(out)

                accs = lax.fori_loop(
                    0, K, body,
                    tuple(jnp.zeros((16,), jnp.float32) for _ in range(G)))
                for g in range(G):
                    o_v[p, pl.ds(g * 16, 16)] = accs[g]
            pltpu.sync_copy(o_v, out_hbm.at[pl.ds(p0, PTS)])

    return sc_layer


_sc_layer1 = _make_sc_layer(128)
_sc_layer2 = _make_sc_layer(64)


def kernel(xyz, features, knn_indices, Wwn1, bwn1, Wlin1, blin1,
           Wwn2, bwn2, Wlin2, blin2, Wfc, bfc):
    pad = NP_PAD - N
    xt = jnp.pad(features[0].T.astype(jnp.float32), ((0, pad), (0, 0)))
    xyzt = jnp.pad(xyz[0].T.astype(jnp.float32), ((0, pad), (0, 5)))
    idx = jnp.pad(knn_indices[0].astype(jnp.int32), ((0, pad), (0, 0)))
    idx = idx.reshape(-1)

    wlin1t = Wlin1.T
    wwn1t = jnp.pad(Wwn1.T, ((0, 5), (0, 0)))
    wwn2t = jnp.pad(Wwn2.T, ((0, 5), (0, 0)))
    wlin2t = Wlin2.T
    wfct = jnp.pad(Wfc.T, ((0, 0), (0, 5)))
    blin1_2d = blin1[None, :]
    bwn1_2d = bwn1[None, :]
    blin2_2d = blin2[None, :]
    bwn2_2d = bwn2[None, :]
    bfc_2d = jnp.pad(bfc, (0, 5))[None, :]

    RB = 2048
    grid = (NP_PAD // RB,)

    def _row(c):
        return pl.BlockSpec((RB, c), lambda i: (i, 0))

    def _full(shape):
        return pl.BlockSpec(shape, lambda i: (0, 0))

    t1, s1, a2t, s2 = pl.pallas_call(
        _prep1_body,
        grid=grid,
        in_specs=[_row(128), _row(8), _full((128, 128)), _full((1, 128)),
                  _full((8, 128)), _full((1, 128)), _full((8, 64)),
                  _full((1, 64))],
        out_specs=[_row(256), _row(128), _row(64), _row(64)],
        out_shape=[
            jax.ShapeDtypeStruct((NP_PAD, 256), jnp.float32),
            jax.ShapeDtypeStruct((NP_PAD, 128), jnp.float32),
            jax.ShapeDtypeStruct((NP_PAD, 64), jnp.float32),
            jax.ShapeDtypeStruct((NP_PAD, 64), jnp.float32),
        ],
    )(xt, xyzt, wlin1t, blin1_2d, wwn1t, bwn1_2d, wwn2t, bwn2_2d)

    out1 = _sc_layer1(t1, idx, s1)

    t2 = pl.pallas_call(
        _prep2_body,
        grid=grid,
        in_specs=[_row(128), _full((128, 64)), _full((1, 64)), _row(64)],
        out_specs=_row(128),
        out_shape=jax.ShapeDtypeStruct((NP_PAD, 128), jnp.float32),
    )(out1, wlin2t, blin2_2d, a2t)

    out2 = _sc_layer2(t2, idx, s2)

    r = pl.pallas_call(
        _final_body,
        grid=grid,
        in_specs=[_row(64), _full((64, 8)), _full((1, 8))],
        out_specs=_row(8),
        out_shape=jax.ShapeDtypeStruct((NP_PAD, 8), jnp.float32),
    )(out2, wfct, bfc_2d)

    return r[:N, :3].T[None]


# speedup vs baseline: 807.4491x; 1.2237x over previous
import functools

import jax
import jax.numpy as jnp
from jax import lax
from jax.experimental import pallas as pl
from jax.experimental.pallas import tpu as pltpu
from jax.experimental.pallas import tpu_sc as plsc

N = 10000
K = 32
NWORK = 32
NP_PAD = 10240
PPW = NP_PAD // NWORK
PTS = 4
EPC = PTS * K
NCHUNK = PPW // PTS


def _lrelu(x):
    return jnp.maximum(x, 0.1 * x)


def _dot(a, b):
    return jnp.dot(a, b, preferred_element_type=jnp.float32,
                   precision=lax.Precision.HIGHEST)


def _prep1_body(xt_ref, xyzt_ref, wlin1t_ref, blin1_ref, wwn1t_ref, bwn1_ref,
                wwn2t_ref, bwn2_ref, t1_ref, s1_ref, a2_ref, s2_ref):
    f1 = _lrelu(_dot(xt_ref[...], wlin1t_ref[...]) + blin1_ref[...])
    a1 = _dot(xyzt_ref[...], wwn1t_ref[...])
    t1_ref[:, :128] = f1 * (1.0 / K)
    t1_ref[:, 128:] = a1
    s1_ref[...] = a1 - bwn1_ref[...]
    a2 = _dot(xyzt_ref[...], wwn2t_ref[...])
    a2_ref[...] = a2
    s2_ref[...] = a2 - bwn2_ref[...]


def _prep2_body(x_ref, wlin2t_ref, blin2_ref, a2_ref, t2_ref):
    f2 = _lrelu(_dot(x_ref[...], wlin2t_ref[...]) + blin2_ref[...])
    t2_ref[:, :64] = f2 * (1.0 / K)
    t2_ref[:, 64:] = a2_ref[...]


def _final_body(x_ref, wfct_ref, bfc_ref, r_ref):
    r_ref[...] = _dot(x_ref[...], wfct_ref[...]) + bfc_ref[...]


CPI = 4
NITER = NCHUNK // CPI
OROWS = CPI * PTS


def _make_sc_layer(C):
    G = C // 16
    mesh = plsc.VectorSubcoreMesh(core_axis_name="c", subcore_axis_name="s")

    @functools.partial(
        pl.kernel,
        mesh=mesh,
        out_type=jax.ShapeDtypeStruct((NP_PAD, C), jnp.float32),
        scratch_types=[
            pltpu.VMEM((PPW * K,), jnp.int32),
            pltpu.VMEM((EPC, 2 * C), jnp.float32),
            pltpu.VMEM((EPC, 2 * C), jnp.float32),
            pltpu.VMEM((PPW, C), jnp.float32),
            pltpu.VMEM((OROWS, C), jnp.float32),
            pltpu.SemaphoreType.DMA,
            pltpu.SemaphoreType.DMA,
            pltpu.SemaphoreType.DMA,
        ],
    )
    def sc_layer(t_hbm, idx_hbm, s_hbm, out_hbm,
                 idx_v, rows0, rows1, s_v, o_v, sem0, sem1, osem):
        wid = lax.axis_index("s") * 2 + lax.axis_index("c")
        base_pt = wid * PPW
        rows = (rows0, rows1)
        sems = (sem0, sem1)

        pltpu.sync_copy(idx_hbm.at[pl.ds(base_pt * K, PPW * K)], idx_v)
        pltpu.sync_copy(s_hbm.at[pl.ds(base_pt, PPW)], s_v)

        def issue(ch, b):
            pltpu.async_copy(
                t_hbm.at[idx_v.at[pl.ds(ch * EPC, EPC)]], rows[b], sems[b])

        def wait(b):
            pltpu.make_async_copy(
                t_hbm.at[idx_v.at[pl.ds(0, EPC)]], rows[b], sems[b]).wait()

        def compute(ch, b, orow):
            for p in range(PTS):
                lp = ch * PTS + p
                svs = [s_v[lp, pl.ds(g * 16, 16)] for g in range(G)]

                def body(k, accs, p=p, svs=svs, b=b):
                    e = p * K + k
                    out = []
                    for g in range(G):
                        a = rows[b][e, pl.ds(C + g * 16, 16)]
                        f = rows[b][e, pl.ds(g * 16, 16)]
                        w = a - svs[g]
                        w = jnp.maximum(w, 0.1 * w)
                        out.append(accs[g] + w * f)
                    return tuple(out)

                accs = lax.fori_loop(
                    0, K, body,
                    tuple(jnp.zeros((16,), jnp.float32) for _ in range(G)))
                for g in range(G):
                    o_v[orow + p, pl.ds(g * 16, 16)] = accs[g]

        issue(0, 0)

        @pl.loop(0, NITER)
        def _it(it):
            c0 = it * CPI

            @pl.when(it > 0)
            def _():
                pltpu.make_async_copy(
                    o_v, out_hbm.at[pl.ds(base_pt, OROWS)], osem).wait()

            issue(c0 + 1, 1)
            wait(0)
            compute(c0, 0, 0)
            issue(c0 + 2, 0)
            wait(1)
            compute(c0 + 1, 1, PTS)
            issue(c0 + 3, 1)
            wait(0)
            compute(c0 + 2, 0, 2 * PTS)

            @pl.when(c0 + CPI < NCHUNK)
            def _():
                issue(c0 + CPI, 0)

            wait(1)
            compute(c0 + 3, 1, 3 * PTS)
            pltpu.async_copy(
                o_v, out_hbm.at[pl.ds(base_pt + it * OROWS, OROWS)], osem)

        pltpu.make_async_copy(
            o_v, out_hbm.at[pl.ds(base_pt, OROWS)], osem).wait()

    return sc_layer


_sc_layer1 = _make_sc_layer(128)
_sc_layer2 = _make_sc_layer(64)


def kernel(xyz, features, knn_indices, Wwn1, bwn1, Wlin1, blin1,
           Wwn2, bwn2, Wlin2, blin2, Wfc, bfc):
    pad = NP_PAD - N
    xt = jnp.pad(features[0].T.astype(jnp.float32), ((0, pad), (0, 0)))
    xyzt = jnp.pad(xyz[0].T.astype(jnp.float32), ((0, pad), (0, 5)))
    idx = jnp.pad(knn_indices[0].astype(jnp.int32), ((0, pad), (0, 0)))
    idx = idx.reshape(-1)

    wlin1t = Wlin1.T
    wwn1t = jnp.pad(Wwn1.T, ((0, 5), (0, 0)))
    wwn2t = jnp.pad(Wwn2.T, ((0, 5), (0, 0)))
    wlin2t = Wlin2.T
    wfct = jnp.pad(Wfc.T, ((0, 0), (0, 5)))
    blin1_2d = blin1[None, :]
    bwn1_2d = bwn1[None, :]
    blin2_2d = blin2[None, :]
    bwn2_2d = bwn2[None, :]
    bfc_2d = jnp.pad(bfc, (0, 5))[None, :]

    RB = 2048
    grid = (NP_PAD // RB,)

    def _row(c):
        return pl.BlockSpec((RB, c), lambda i: (i, 0))

    def _full(shape):
        return pl.BlockSpec(shape, lambda i: (0, 0))

    t1, s1, a2t, s2 = pl.pallas_call(
        _prep1_body,
        grid=grid,
        in_specs=[_row(128), _row(8), _full((128, 128)), _full((1, 128)),
                  _full((8, 128)), _full((1, 128)), _full((8, 64)),
                  _full((1, 64))],
        out_specs=[_row(256), _row(128), _row(64), _row(64)],
        out_shape=[
            jax.ShapeDtypeStruct((NP_PAD, 256), jnp.float32),
            jax.ShapeDtypeStruct((NP_PAD, 128), jnp.float32),
            jax.ShapeDtypeStruct((NP_PAD, 64), jnp.float32),
            jax.ShapeDtypeStruct((NP_PAD, 64), jnp.float32),
        ],
    )(xt, xyzt, wlin1t, blin1_2d, wwn1t, bwn1_2d, wwn2t, bwn2_2d)

    out1 = _sc_layer1(t1, idx, s1)

    t2 = pl.pallas_call(
        _prep2_body,
        grid=grid,
        in_specs=[_row(128), _full((128, 64)), _full((1, 64)), _row(64)],
        out_specs=_row(128),
        out_shape=jax.ShapeDtypeStruct((NP_PAD, 128), jnp.float32),
    )(out1, wlin2t, blin2_2d, a2t)

    out2 = _sc_layer2(t2, idx, s2)

    r = pl.pallas_call(
        _final_body,
        grid=grid,
        in_specs=[_row(64), _full((64, 8)), _full((1, 8))],
        out_specs=_row(8),
        out_shape=jax.ShapeDtypeStruct((NP_PAD, 8), jnp.float32),
    )(out2, wfct, bfc_2d)

    return r[:N, :3].T[None]


# speedup vs baseline: 964.2358x; 1.1942x over previous
import functools

import jax
import jax.numpy as jnp
from jax import lax
from jax.experimental import pallas as pl
from jax.experimental.pallas import tpu as pltpu
from jax.experimental.pallas import tpu_sc as plsc

N = 10000
K = 32
NWORK = 32
NP_PAD = 10240
PPW = NP_PAD // NWORK
PTS = 4
EPC = PTS * K
NCHUNK = PPW // PTS


def _lrelu(x):
    return jnp.maximum(x, 0.1 * x)


def _dot(a, b):
    return jnp.dot(a, b, preferred_element_type=jnp.float32,
                   precision=lax.Precision.HIGHEST)


def _prep1_body(xt_ref, xyzt_ref, wlin1t_ref, blin1_ref, wwn1t_ref, bwn1_ref,
                wwn2t_ref, bwn2_ref, t1_ref, s1_ref, a2_ref, s2_ref):
    f1 = _lrelu(_dot(xt_ref[...], wlin1t_ref[...]) + blin1_ref[...])
    a1 = _dot(xyzt_ref[...], wwn1t_ref[...])
    t1_ref[:, :128] = f1 * (1.0 / K)
    t1_ref[:, 128:] = a1
    s1_ref[...] = a1 - bwn1_ref[...]
    a2 = _dot(xyzt_ref[...], wwn2t_ref[...])
    a2_ref[...] = a2
    s2_ref[...] = a2 - bwn2_ref[...]


def _prep2_body(x_ref, wlin2t_ref, blin2_ref, a2_ref, t2_ref):
    f2 = _lrelu(_dot(x_ref[...], wlin2t_ref[...]) + blin2_ref[...])
    t2_ref[:, :64] = f2 * (1.0 / K)
    t2_ref[:, 64:] = a2_ref[...]


def _final_body(x_ref, wfct_ref, bfc_ref, r_ref):
    r_ref[...] = _dot(x_ref[...], wfct_ref[...]) + bfc_ref[...]


PTS = 2
EPC = PTS * K
CPI = 8
OROWS = CPI * PTS
PPW0 = 496
PPW1 = 144
PPW_MAX = max(PPW0, PPW1)
NP_BIG = 11264


def _make_sc_layer(C):
    G = C // 16
    mesh = plsc.VectorSubcoreMesh(core_axis_name="c", subcore_axis_name="s")

    @functools.partial(
        pl.kernel,
        mesh=mesh,
        out_type=jax.ShapeDtypeStruct((NP_PAD, C), jnp.float32),
        scratch_types=[
            pltpu.VMEM((PPW_MAX * K,), jnp.int32),
            pltpu.VMEM((EPC, 2 * C), jnp.float32),
            pltpu.VMEM((EPC, 2 * C), jnp.float32),
            pltpu.VMEM((PPW_MAX, C), jnp.float32),
            pltpu.VMEM((OROWS, C), jnp.float32),
            pltpu.SemaphoreType.DMA,
            pltpu.SemaphoreType.DMA,
            pltpu.SemaphoreType.DMA,
        ],
    )
    def sc_layer(t_hbm, idx_hbm, s_hbm, out_hbm,
                 idx_v, rows0, rows1, s_v, o_v, sem0, sem1, osem):
        c = lax.axis_index("c")
        s = lax.axis_index("s")
        base_pt = jnp.where(c == 0, s * PPW0, 16 * PPW0 + s * PPW1)
        niter = jnp.where(c == 0, PPW0 // OROWS, PPW1 // OROWS)
        rows = (rows0, rows1)
        sems = (sem0, sem1)

        pltpu.sync_copy(idx_hbm.at[pl.ds(base_pt * K, PPW_MAX * K)], idx_v)
        pltpu.sync_copy(s_hbm.at[pl.ds(base_pt, PPW_MAX)], s_v)

        def issue(ch, b):
            pltpu.async_copy(
                t_hbm.at[idx_v.at[pl.ds(ch * EPC, EPC)]], rows[b], sems[b])

        def wait(b):
            pltpu.make_async_copy(
                t_hbm.at[idx_v.at[pl.ds(0, EPC)]], rows[b], sems[b]).wait()

        def compute(ch, b, orow):
            for p in range(PTS):
                lp = ch * PTS + p
                svs = [s_v[lp, pl.ds(g * 16, 16)] for g in range(G)]

                def body(k, accs, p=p, svs=svs, b=b):
                    e = p * K + k
                    out = []
                    for g in range(G):
                        a = rows[b][e, pl.ds(C + g * 16, 16)]
                        f = rows[b][e, pl.ds(g * 16, 16)]
                        w = a - svs[g]
                        w = jnp.maximum(w, 0.1 * w)
                        out.append(accs[g] + w * f)
                    return tuple(out)

                accs = lax.fori_loop(
                    0, K, body,
                    tuple(jnp.zeros((16,), jnp.float32) for _ in range(G)))
                for g in range(G):
                    o_v[orow + p, pl.ds(g * 16, 16)] = accs[g]

        issue(0, 0)

        @pl.loop(0, niter)
        def _it(it):
            c0 = it * CPI

            @pl.when(it > 0)
            def _():
                pltpu.make_async_copy(
                    o_v, out_hbm.at[pl.ds(base_pt, OROWS)], osem).wait()

            for j in range(CPI):
                if j == CPI - 1:
                    @pl.when(it + 1 < niter)
                    def _():
                        issue(c0 + CPI, 0)
                else:
                    issue(c0 + j + 1, (j + 1) % 2)
                wait(j % 2)
                compute(c0 + j, j % 2, j * PTS)

            pltpu.async_copy(
                o_v, out_hbm.at[pl.ds(base_pt + it * OROWS, OROWS)], osem)

        pltpu.make_async_copy(
            o_v, out_hbm.at[pl.ds(base_pt, OROWS)], osem).wait()

    return sc_layer


_sc_layer1 = _make_sc_layer(128)
_sc_layer2 = _make_sc_layer(64)


def kernel(xyz, features, knn_indices, Wwn1, bwn1, Wlin1, blin1,
           Wwn2, bwn2, Wlin2, blin2, Wfc, bfc):
    pad = NP_BIG - N
    xt = jnp.pad(features[0].T.astype(jnp.float32), ((0, pad), (0, 0)))
    xyzt = jnp.pad(xyz[0].T.astype(jnp.float32), ((0, pad), (0, 5)))
    idx = jnp.pad(knn_indices[0].astype(jnp.int32), ((0, pad), (0, 0)))
    idx = idx.reshape(-1)

    wlin1t = Wlin1.T
    wwn1t = jnp.pad(Wwn1.T, ((0, 5), (0, 0)))
    wwn2t = jnp.pad(Wwn2.T, ((0, 5), (0, 0)))
    wlin2t = Wlin2.T
    wfct = jnp.pad(Wfc.T, ((0, 0), (0, 5)))
    blin1_2d = blin1[None, :]
    bwn1_2d = bwn1[None, :]
    blin2_2d = blin2[None, :]
    bwn2_2d = bwn2[None, :]
    bfc_2d = jnp.pad(bfc, (0, 5))[None, :]

    RB = 1024
    grid_big = (NP_BIG // RB,)
    grid = (NP_PAD // RB,)

    def _row(c):
        return pl.BlockSpec((RB, c), lambda i: (i, 0))

    def _full(shape):
        return pl.BlockSpec(shape, lambda i: (0, 0))

    t1, s1, a2t, s2 = pl.pallas_call(
        _prep1_body,
        grid=grid_big,
        in_specs=[_row(128), _row(8), _full((128, 128)), _full((1, 128)),
                  _full((8, 128)), _full((1, 128)), _full((8, 64)),
                  _full((1, 64))],
        out_specs=[_row(256), _row(128), _row(64), _row(64)],
        out_shape=[
            jax.ShapeDtypeStruct((NP_BIG, 256), jnp.float32),
            jax.ShapeDtypeStruct((NP_BIG, 128), jnp.float32),
            jax.ShapeDtypeStruct((NP_BIG, 64), jnp.float32),
            jax.ShapeDtypeStruct((NP_BIG, 64), jnp.float32),
        ],
    )(xt, xyzt, wlin1t, blin1_2d, wwn1t, bwn1_2d, wwn2t, bwn2_2d)

    out1 = _sc_layer1(t1, idx, s1)

    t2 = pl.pallas_call(
        _prep2_body,
        grid=grid,
        in_specs=[_row(128), _full((128, 64)), _full((1, 64)), _row(64)],
        out_specs=_row(128),
        out_shape=jax.ShapeDtypeStruct((NP_PAD, 128), jnp.float32),
    )(out1, wlin2t, blin2_2d, a2t)

    out2 = _sc_layer2(t2, idx, s2)

    r = pl.pallas_call(
        _final_body,
        grid=grid,
        in_specs=[_row(64), _full((64, 8)), _full((1, 8))],
        out_specs=_row(8),
        out_shape=jax.ShapeDtypeStruct((NP_PAD, 8), jnp.float32),
    )(out2, wfct, bfc_2d)

    return r[:N, :3].T[None]
